# Initial kernel scaffold; baseline (speedup 1.0000x reference)
import jax
import jax.numpy as jnp
from jax.experimental import pallas as pl


def kernel(x, emb_weight):
    raise NotImplementedError("write your pallas kernel here")


# speedup vs baseline: 1.3728x; 1.3728x over previous
import functools

import jax
import jax.numpy as jnp
from jax import lax
from jax.experimental import pallas as pl
from jax.experimental.pallas import tpu as pltpu
from jax.experimental.pallas import tpu_sc as plsc

_B = 100000
_D = 256
_C = 128
_NW = 32
_NCHUNK = (_B + _C - 1) // _C
_LAST = _B - _C
_ITERS = -(-_NCHUNK // _NW)

_mesh = plsc.VectorSubcoreMesh(core_axis_name="c", subcore_axis_name="s")


@functools.partial(
    pl.kernel,
    mesh=_mesh,
    out_type=jax.ShapeDtypeStruct((_B, _D), jnp.float32),
    scratch_types=[
        pltpu.VMEM((_C,), jnp.int32),
        pltpu.VMEM((_C,), jnp.int32),
        pltpu.VMEM((_C,), jnp.int32),
        pltpu.VMEM((_C, _D), jnp.float32),
        pltpu.VMEM((_C, _D), jnp.float32),
        pltpu.VMEM((_C, _D), jnp.float32),
        pltpu.SemaphoreType.DMA,
        pltpu.SemaphoreType.DMA,
        pltpu.SemaphoreType.DMA,
        pltpu.SemaphoreType.DMA,
        pltpu.SemaphoreType.DMA,
        pltpu.SemaphoreType.DMA,
    ],
)
def _emb_lookup(idx_hbm, table_hbm, out_hbm,
                i0, i1, i2, r0, r1, r2,
                g0, g1, g2, o0, o1, o2):
    w = lax.axis_index("s") * 2 + lax.axis_index("c")
    idx_bufs = (i0, i1, i2)
    row_bufs = (r0, r1, r2)
    gsems = (g0, g1, g2)
    osems = (o0, o1, o2)

    def chunk_start(i):
        return jnp.minimum((w + _NW * i) * _C, _LAST)

    def fire_gather(i, s):
        st = chunk_start(i)
        pltpu.sync_copy(idx_hbm.at[pl.ds(st, _C)], idx_bufs[s])
        pltpu.async_copy(table_hbm.at[idx_bufs[s]], row_bufs[s], gsems[s])

    def wait_gather(s):
        pltpu.make_async_copy(
            table_hbm.at[idx_bufs[s]], row_bufs[s], gsems[s]).wait()

    def fire_out(i, s):
        st = chunk_start(i)
        pltpu.async_copy(row_bufs[s], out_hbm.at[pl.ds(st, _C)], osems[s])

    def wait_out(i, s):
        st = chunk_start(i)
        pltpu.make_async_copy(
            row_bufs[s], out_hbm.at[pl.ds(st, _C)], osems[s]).wait()

    fire_gather(0, 0)
    fire_gather(1, 1)
    wait_gather(0)
    fire_out(0, 0)
    fire_gather(2, 2)
    wait_gather(1)
    fire_out(1, 1)

    def body(j, carry):
        base = 3 * j
        for t in range(3):
            i = base + t
            wait_out(i - 3, t)
            fire_gather(i, t)
            sp = (t + 2) % 3
            wait_gather(sp)
            fire_out(i - 1, sp)
        return carry

    lax.fori_loop(1, (_ITERS - 1) // 3, body, 0)

    wait_out(_ITERS - 4, 0)
    fire_gather(_ITERS - 1, 0)
    wait_gather(2)
    fire_out(_ITERS - 2, 2)
    wait_gather(0)
    fire_out(_ITERS - 1, 0)
    wait_out(_ITERS - 3, 1)
    wait_out(_ITERS - 2, 2)
    wait_out(_ITERS - 1, 0)


def kernel(x, emb_weight):
    return _emb_lookup(x.astype(jnp.int32), emb_weight)
